# Initial kernel scaffold; baseline (speedup 1.0000x reference)
import jax
import jax.numpy as jnp
from jax.experimental import pallas as pl


def kernel(x, edge_index, W1, b1, W2, b2):
    raise NotImplementedError("write your pallas kernel here")


# speedup vs baseline: 6.5871x; 6.5871x over previous
import functools

import jax
import jax.numpy as jnp
from jax import lax
from jax.experimental import pallas as pl
from jax.experimental.pallas import tpu as pltpu
from jax.experimental.pallas import tpu_sc as plsc

NC, NS, LANES = 2, 16, 16
NW = NC * NS

N = 10000
NE = 10000
E = 160000
NP = 10240
CHUNK = 128
CHUNKS = 40
EP = NW * CHUNKS * CHUNK
ROWS_W = NP // NS


def _matmul_kernel(x_ref, w_ref, o_ref):
    o_ref[...] = jnp.dot(x_ref[...], w_ref[...],
                         preferred_element_type=jnp.float32)


def _matmul(x, w):
    M, K = x.shape
    D = w.shape[1]
    return pl.pallas_call(
        _matmul_kernel,
        out_shape=jax.ShapeDtypeStruct((M, D), jnp.float32),
    )(x, w)


def _scale_kernel(accp_ref, degp_ref, o_ref):
    d = degp_ref[0] + degp_ref[1]
    dinv = jnp.where(d > 0.0, 1.0 / d, 0.0)
    o_ref[...] = (accp_ref[0] + accp_ref[1]) * dinv[:, None]


def _scale(acc_p, deg_p):
    _, M, D = acc_p.shape
    return pl.pallas_call(
        _scale_kernel,
        out_shape=jax.ShapeDtypeStruct((M, D), jnp.float32),
    )(acc_p, deg_p)


def _finish1_kernel(accp_ref, degp_ref, b1_ref, w2_ref, o_ref):
    d = degp_ref[0] + degp_ref[1]
    dinv = jnp.where(d > 0.0, 1.0 / d, 0.0)
    h = (accp_ref[0] + accp_ref[1]) * dinv[:, None] + b1_ref[...]
    h = jnp.maximum(h, 0.0)
    o_ref[...] = jnp.dot(h, w2_ref[...], preferred_element_type=jnp.float32)


def _finish1(acc_p, deg_p, b1, w2p):
    _, M, _ = acc_p.shape
    D = w2p.shape[1]
    return pl.pallas_call(
        _finish1_kernel,
        out_shape=jax.ShapeDtypeStruct((M, D), jnp.float32),
    )(acc_p, deg_p, b1, w2p)


def _finish2_kernel(accp_ref, degp_ref, b2_ref, o_ref):
    d = degp_ref[0] + degp_ref[1]
    dinv = jnp.where(d > 0.0, 1.0 / d, 0.0)
    o_ref[...] = (accp_ref[0] + accp_ref[1]) * dinv[:, None] + b2_ref[...]


def _finish2(acc_p, deg_p, b2p):
    _, M, D = acc_p.shape
    return pl.pallas_call(
        _finish2_kernel,
        out_shape=jax.ShapeDtypeStruct((M, D), jnp.float32),
    )(acc_p, deg_p, b2p)


def _sc_pass(table, gidx_all, sidx_all, D, with_deg):
    outs = [jax.ShapeDtypeStruct((NC * NP, D), jnp.float32)]
    if with_deg:
        outs.append(jax.ShapeDtypeStruct((NC * NP,), jnp.float32))
        outs.append(jax.ShapeDtypeStruct((NC * NP,), jnp.float32))
    scratch = [
        pltpu.VMEM((CHUNK,), jnp.int32),
        pltpu.VMEM((CHUNK,), jnp.int32),
        pltpu.VMEM((CHUNK, D), jnp.float32),
        pltpu.VMEM((ROWS_W, D), jnp.float32),
        pltpu.VMEM_SHARED((NP, D), jnp.float32),
        pltpu.SemaphoreType.DMA,
    ]
    if with_deg:
        scratch += [
            pltpu.VMEM((CHUNK,), jnp.float32),
            pltpu.VMEM((ROWS_W,), jnp.float32),
            pltpu.VMEM_SHARED((NP,), jnp.float32),
            pltpu.VMEM_SHARED((NP,), jnp.float32),
        ]
    mesh = plsc.VectorSubcoreMesh(core_axis_name="c", subcore_axis_name="s")

    @functools.partial(
        pl.kernel, out_type=tuple(outs), mesh=mesh, scratch_types=scratch,
        compiler_params=pltpu.CompilerParams(use_tc_tiling_on_sc=False))
    def run(table_h, gi_h, si_h, *refs):
        if with_deg:
            (acc_h, dg_h, db_h, gidx_v, sidx_v, rows_v, zbuf_v, acc_sp,
             sem, ones_v, zvec_v, dg_sp, db_sp) = refs
        else:
            (acc_h, gidx_v, sidx_v, rows_v, zbuf_v, acc_sp, sem) = refs
        sid = lax.axis_index("s")
        cid = lax.axis_index("c")
        gw = sid * NC + cid
        r0 = sid * ROWS_W

        def zero_row(i, _):
            for j in range(D // LANES):
                zbuf_v[i, pl.ds(j * LANES, LANES)] = jnp.zeros(
                    (LANES,), jnp.float32)
            return 0

        lax.fori_loop(0, ROWS_W, zero_row, 0)
        pltpu.sync_copy(zbuf_v, acc_sp.at[pl.ds(r0, ROWS_W)])

        if with_deg:
            def fill_small(i, _):
                zvec_v[pl.ds(i * LANES, LANES)] = jnp.zeros(
                    (LANES,), jnp.float32)
                ones_v[pl.ds((i % (CHUNK // LANES)) * LANES, LANES)] = (
                    jnp.ones((LANES,), jnp.float32))
                return 0

            lax.fori_loop(0, ROWS_W // LANES, fill_small, 0)
            pltpu.sync_copy(zvec_v, dg_sp.at[pl.ds(r0, ROWS_W)])
            pltpu.sync_copy(zvec_v, db_sp.at[pl.ds(r0, ROWS_W)])

        plsc.subcore_barrier()

        def chunk_body(ci, _):
            off = (gw * CHUNKS + ci) * CHUNK
            pltpu.sync_copy(gi_h.at[pl.ds(off, CHUNK)], gidx_v)
            pltpu.sync_copy(si_h.at[pl.ds(off, CHUNK)], sidx_v)
            pltpu.async_copy(table_h.at[gidx_v], rows_v, sem).wait()
            pltpu.sync_copy(rows_v, acc_sp.at[sidx_v], add=True)
            if with_deg:
                pltpu.sync_copy(ones_v, dg_sp.at[gidx_v], add=True)
                pltpu.sync_copy(ones_v, db_sp.at[sidx_v], add=True)
            return 0

        lax.fori_loop(0, CHUNKS, chunk_body, 0)
        plsc.subcore_barrier()

        o0 = cid * NP + r0
        pltpu.sync_copy(acc_sp.at[pl.ds(r0, ROWS_W)],
                        acc_h.at[pl.ds(o0, ROWS_W)])
        if with_deg:
            pltpu.sync_copy(dg_sp.at[pl.ds(r0, ROWS_W)],
                            dg_h.at[pl.ds(o0, ROWS_W)])
            pltpu.sync_copy(db_sp.at[pl.ds(r0, ROWS_W)],
                            db_h.at[pl.ds(o0, ROWS_W)])

    res = run(table, gidx_all, sidx_all)
    if with_deg:
        acc, dg, db = res
        return (acc.reshape(NC, NP, D), dg.reshape(NC, NP),
                db.reshape(NC, NP))
    return res[0].reshape(NC, NP, D)


def kernel(x, edge_index, W1, b1, W2, b2):
    hid = W1.shape[1]
    out_d = W2.shape[1]
    d2p = LANES

    x_pad = jnp.concatenate(
        [x, jnp.zeros((NP - N, x.shape[1]), jnp.float32)], axis=0)
    src = edge_index[0]
    dst = edge_index[1]
    pad_ids = jnp.arange(EP - E, dtype=jnp.int32) % 16
    src_p = jnp.concatenate([src, N + pad_ids])
    dst_p = jnp.concatenate([dst, NE + pad_ids])
    w2p = jnp.concatenate(
        [W2, jnp.zeros((hid, d2p - out_d), jnp.float32)], axis=1)
    b1r = b1.reshape(1, hid)
    b2r = jnp.concatenate([b2, jnp.zeros((d2p - out_d,), jnp.float32)]
                          ).reshape(1, d2p)

    xw = _matmul(x_pad, W1)
    he_p, deg_p, bdeg_p = _sc_pass(xw, src_p, dst_p, hid, True)
    he = _scale(he_p, bdeg_p)
    out1_p = _sc_pass(he, dst_p, src_p, hid, False)
    hw2 = _finish1(out1_p, deg_p, b1r, w2p)

    he2_p = _sc_pass(hw2, src_p, dst_p, d2p, False)
    he2 = _scale(he2_p, bdeg_p)
    out2_p = _sc_pass(he2, dst_p, src_p, d2p, False)
    out_full = _finish2(out2_p, deg_p, b2r)

    return out_full[:N, :out_d]
